# Initial kernel scaffold; baseline (speedup 1.0000x reference)
import jax
import jax.numpy as jnp
from jax.experimental import pallas as pl


def kernel(input_, weight):
    raise NotImplementedError("write your pallas kernel here")


# speedup vs baseline: 1.8178x; 1.8178x over previous
import functools

import jax
import jax.numpy as jnp
from jax import lax
from jax.experimental import pallas as pl
from jax.experimental.pallas import tpu as pltpu
from jax.experimental.pallas import tpu_sc as plsc

_NUM_EMBEDDINGS = 1000000
_EMBEDDING_DIM = 64
_B = 16384 * 50

_info = plsc.get_sparse_core_info()
_NC, _NS = _info.num_cores, _info.num_subcores
_NW = _NC * _NS
_B_PER_W = _B // _NW
_CHUNK = 640
_GATHER = 128
_N_CHUNKS = _B_PER_W // _CHUNK


def _body(table_hbm, idx_hbm, out_hbm, idx_v, rows_v, sem):
    wid = lax.axis_index("s") * _NC + lax.axis_index("c")
    base = wid * _B_PER_W

    def chunk(i, carry):
        off = base + i * _CHUNK
        pltpu.sync_copy(idx_hbm.at[pl.ds(off, _CHUNK)], idx_v)
        copies = []
        for j in range(_CHUNK // _GATHER):
            copies.append(
                pltpu.async_copy(
                    table_hbm.at[idx_v.at[pl.ds(j * _GATHER, _GATHER)]],
                    rows_v.at[pl.ds(j * _GATHER, _GATHER)],
                    sem,
                )
            )
        for c in copies:
            c.wait()
        pltpu.sync_copy(rows_v, out_hbm.at[pl.ds(off, _CHUNK)])
        return carry

    lax.fori_loop(0, _N_CHUNKS, chunk, 0)


@functools.partial(jax.jit, static_argnames=())
def _lookup(weight, idx):
    mesh = plsc.VectorSubcoreMesh(core_axis_name="c", subcore_axis_name="s")
    f = pl.kernel(
        _body,
        mesh=mesh,
        out_type=jax.ShapeDtypeStruct((_B, _EMBEDDING_DIM), jnp.float32),
        scratch_types=[
            pltpu.VMEM((_CHUNK,), jnp.int32),
            pltpu.VMEM((_CHUNK, _EMBEDDING_DIM), jnp.float32),
            pltpu.SemaphoreType.DMA,
        ],
        compiler_params=pltpu.CompilerParams(use_tc_tiling_on_sc=False),
    )
    return f(weight, idx)


def kernel(input_, weight):
    idx = input_.reshape(-1).astype(jnp.int32)
    out = _lookup(weight, idx)
    return out.reshape(input_.shape + (weight.shape[-1],))


# speedup vs baseline: 1.8753x; 1.0317x over previous
import functools

import jax
import jax.numpy as jnp
from jax import lax
from jax.experimental import pallas as pl
from jax.experimental.pallas import tpu as pltpu
from jax.experimental.pallas import tpu_sc as plsc

_NUM_EMBEDDINGS = 1000000
_EMBEDDING_DIM = 64
_B = 16384 * 50

_info = plsc.get_sparse_core_info()
_NC, _NS = _info.num_cores, _info.num_subcores
_NW = _NC * _NS
_B_PER_W = _B // _NW
_CHUNK = 640
_GATHER = 128
_N_CHUNKS = _B_PER_W // _CHUNK
_NBUF = 2


def _body(table_hbm, idx_hbm, out_hbm,
          idx0, idx1, rows0, rows1,
          idx_sem0, idx_sem1, gat_sem0, gat_sem1, out_sem0, out_sem1):
    wid = lax.axis_index("s") * _NC + lax.axis_index("c")
    base = wid * _B_PER_W
    idx_v = (idx0, idx1)
    rows_v = (rows0, rows1)
    idx_sem = (idx_sem0, idx_sem1)
    gat_sem = (gat_sem0, gat_sem1)
    out_sem = (out_sem0, out_sem1)

    def start_idx(b, i):
        pltpu.async_copy(
            idx_hbm.at[pl.ds(base + i * _CHUNK, _CHUNK)], idx_v[b], idx_sem[b])

    def wait_idx(b, i):
        pltpu.make_async_copy(
            idx_hbm.at[pl.ds(base + i * _CHUNK, _CHUNK)], idx_v[b],
            idx_sem[b]).wait()

    def gather(b):
        for j in range(_CHUNK // _GATHER):
            pltpu.async_copy(
                table_hbm.at[idx_v[b].at[pl.ds(j * _GATHER, _GATHER)]],
                rows_v[b].at[pl.ds(j * _GATHER, _GATHER)], gat_sem[b])
        for j in range(_CHUNK // _GATHER):
            pltpu.make_async_copy(
                table_hbm.at[idx_v[b].at[pl.ds(0, _GATHER)]],
                rows_v[b].at[pl.ds(0, _GATHER)], gat_sem[b]).wait()

    def start_out(b, i):
        pltpu.async_copy(
            rows_v[b], out_hbm.at[pl.ds(base + i * _CHUNK, _CHUNK)], out_sem[b])

    def wait_out(b, i):
        pltpu.make_async_copy(
            rows_v[b], out_hbm.at[pl.ds(base + i * _CHUNK, _CHUNK)],
            out_sem[b]).wait()

    for b in range(_NBUF):
        start_idx(b, b)
    for b in range(_NBUF):
        wait_idx(b, b)
        gather(b)
        start_out(b, b)
        start_idx(b, b + _NBUF)

    def step(g, carry):
        for b in range(_NBUF):
            i = g * _NBUF + b
            wait_idx(b, i)
            wait_out(b, i)
            gather(b)
            start_out(b, i)
            start_idx(b, i + _NBUF)
        return carry

    lax.fori_loop(1, _N_CHUNKS // _NBUF - 1, step, 0, unroll=False)

    for b in range(_NBUF):
        i = _N_CHUNKS - _NBUF + b
        wait_idx(b, i)
        wait_out(b, i)
        gather(b)
        start_out(b, i)
    for b in range(_NBUF):
        wait_out(b, _N_CHUNKS - _NBUF + b)


@jax.jit
def _lookup(weight, idx):
    mesh = plsc.VectorSubcoreMesh(core_axis_name="c", subcore_axis_name="s")
    f = pl.kernel(
        _body,
        mesh=mesh,
        out_type=jax.ShapeDtypeStruct((_B, _EMBEDDING_DIM), jnp.float32),
        scratch_types=[
            pltpu.VMEM((_CHUNK,), jnp.int32),
            pltpu.VMEM((_CHUNK,), jnp.int32),
            pltpu.VMEM((_CHUNK, _EMBEDDING_DIM), jnp.float32),
            pltpu.VMEM((_CHUNK, _EMBEDDING_DIM), jnp.float32),
            pltpu.SemaphoreType.DMA,
            pltpu.SemaphoreType.DMA,
            pltpu.SemaphoreType.DMA,
            pltpu.SemaphoreType.DMA,
            pltpu.SemaphoreType.DMA,
            pltpu.SemaphoreType.DMA,
        ],
        compiler_params=pltpu.CompilerParams(use_tc_tiling_on_sc=False),
    )
    return f(weight, idx)


def kernel(input_, weight):
    idx = input_.reshape(-1).astype(jnp.int32)
    out = _lookup(weight, idx)
    return out.reshape(input_.shape + (weight.shape[-1],))
